# Initial kernel scaffold; baseline (speedup 1.0000x reference)
import jax
import jax.numpy as jnp
from jax.experimental import pallas as pl


def kernel(x, w1, b1, w2, b2):
    raise NotImplementedError("write your pallas kernel here")


# speedup vs baseline: 1.6818x; 1.6818x over previous
import jax
import jax.numpy as jnp
from jax.experimental import pallas as pl
from jax.experimental.pallas import tpu as pltpu


def _ffn_kernel(x_ref, w1_ref, b1_ref, w2_ref, b2_ref, o_ref):
    xb = x_ref[...].astype(jnp.bfloat16)
    h = jnp.dot(xb, w1_ref[...], preferred_element_type=jnp.float32)
    h = jnp.maximum(h + b1_ref[...], 0.0)
    y = jnp.dot(h.astype(jnp.bfloat16), w2_ref[...],
                preferred_element_type=jnp.float32)
    o_ref[...] = y + b2_ref[...]


def kernel(x, w1, b1, w2, b2):
    B, S, D = x.shape
    F = w1.shape[1]
    R = B * S
    TM = 512

    x2 = x.reshape(R, D)
    w1b = w1.astype(jnp.bfloat16)
    w2b = w2.astype(jnp.bfloat16)

    out = pl.pallas_call(
        _ffn_kernel,
        out_shape=jax.ShapeDtypeStruct((R, D), x.dtype),
        grid=(pl.cdiv(R, TM),),
        in_specs=[
            pl.BlockSpec((TM, D), lambda i: (i, 0)),
            pl.BlockSpec((D, F), lambda i: (0, 0)),
            pl.BlockSpec((1, F), lambda i: (0, 0)),
            pl.BlockSpec((F, D), lambda i: (0, 0)),
            pl.BlockSpec((1, D), lambda i: (0, 0)),
        ],
        out_specs=pl.BlockSpec((TM, D), lambda i: (i, 0)),
        compiler_params=pltpu.CompilerParams(
            dimension_semantics=("parallel",),
            vmem_limit_bytes=60 * 1024 * 1024,
        ),
        cost_estimate=pl.CostEstimate(
            flops=4 * R * D * F,
            transcendentals=0,
            bytes_accessed=2 * R * D * 4 + 2 * D * F * 2 + F * 4 + D * 4,
        ),
    )(x2, w1b, b1.reshape(1, F), w2b, b2.reshape(1, D))
    return out.reshape(B, S, D)


# speedup vs baseline: 1.7016x; 1.0118x over previous
import jax
import jax.numpy as jnp
from jax.experimental import pallas as pl
from jax.experimental.pallas import tpu as pltpu


def _ffn_kernel(x_ref, w1_ref, b1_ref, w2_ref, b2_ref, o_ref):
    xb = x_ref[...].astype(jnp.bfloat16)
    h = jnp.dot(xb, w1_ref[...], preferred_element_type=jnp.float32)
    h = jnp.maximum(h + b1_ref[...], 0.0)
    y = jnp.dot(h.astype(jnp.bfloat16), w2_ref[...],
                preferred_element_type=jnp.float32)
    o_ref[...] = y + b2_ref[...]


def kernel(x, w1, b1, w2, b2):
    B, S, D = x.shape
    F = w1.shape[1]
    R = B * S
    TM = 1024

    x2 = x.reshape(R, D)
    w1b = w1.astype(jnp.bfloat16)
    w2b = w2.astype(jnp.bfloat16)

    out = pl.pallas_call(
        _ffn_kernel,
        out_shape=jax.ShapeDtypeStruct((R, D), x.dtype),
        grid=(pl.cdiv(R, TM),),
        in_specs=[
            pl.BlockSpec((TM, D), lambda i: (i, 0)),
            pl.BlockSpec((D, F), lambda i: (0, 0)),
            pl.BlockSpec((1, F), lambda i: (0, 0)),
            pl.BlockSpec((F, D), lambda i: (0, 0)),
            pl.BlockSpec((1, D), lambda i: (0, 0)),
        ],
        out_specs=pl.BlockSpec((TM, D), lambda i: (i, 0)),
        compiler_params=pltpu.CompilerParams(
            dimension_semantics=("arbitrary",),
            vmem_limit_bytes=60 * 1024 * 1024,
        ),
        cost_estimate=pl.CostEstimate(
            flops=4 * R * D * F,
            transcendentals=0,
            bytes_accessed=2 * R * D * 4 + 2 * D * F * 2 + F * 4 + D * 4,
        ),
    )(x2, w1b, b1.reshape(1, F), w2b, b2.reshape(1, D))
    return out.reshape(B, S, D)


# speedup vs baseline: 1.7573x; 1.0327x over previous
import jax
import jax.numpy as jnp
from jax.experimental import pallas as pl
from jax.experimental.pallas import tpu as pltpu


def _ffn_body(x_ref, w1_ref, b1_ref, w2_ref, b2_ref, o_ref):
    h = jnp.dot(x_ref[...], w1_ref[...], preferred_element_type=jnp.float32)
    h = jnp.maximum(h + b1_ref[...], 0.0)
    o_ref[...] = b2_ref[...] + jnp.dot(
        h, w2_ref[...], preferred_element_type=jnp.float32)


def kernel(x, w1, b1, w2, b2):
    B, S, D = x.shape
    F = w1.shape[1]
    R = B * S
    TM = 512

    out = pl.pallas_call(
        _ffn_body,
        out_shape=jax.ShapeDtypeStruct((R, D), x.dtype),
        grid=(pl.cdiv(R, TM),),
        in_specs=[
            pl.BlockSpec((TM, D), lambda i: (i, 0)),
            pl.BlockSpec((D, F), lambda i: (0, 0)),
            pl.BlockSpec((1, F), lambda i: (0, 0)),
            pl.BlockSpec((F, D), lambda i: (0, 0)),
            pl.BlockSpec((1, D), lambda i: (0, 0)),
        ],
        out_specs=pl.BlockSpec((TM, D), lambda i: (i, 0)),
        compiler_params=pltpu.CompilerParams(
            dimension_semantics=("arbitrary",),
            vmem_limit_bytes=60 * 1024 * 1024,
        ),
        cost_estimate=pl.CostEstimate(
            flops=4 * R * D * F,
            transcendentals=0,
            bytes_accessed=(2 * R * D + 2 * D * F + F + D) * 4,
        ),
    )(x.reshape(R, D), w1, b1.reshape(1, F), w2, b2.reshape(1, D))
    return out.reshape(B, S, D)
